# baseline (speedup 1.0000x reference)
import functools

import jax
import jax.numpy as jnp
from jax import lax
from jax.experimental import pallas as pl
from jax.experimental.pallas import tpu as pltpu
from jax.experimental.pallas import tpu_sc as plsc

NN = 50000
EE = 800000
NC = 2
NS = 16
NW = NC * NS
SLAB = 1536
NSLAB = 17
CH = SLAB * NSLAB
EP = CH * NW
NPD = 53248
SUBSEG = NPD // NS
G = 128

_f32 = jnp.float32
_i32 = jnp.int32


def _mesh():
  return plsc.VectorSubcoreMesh(
      core_axis_name="c", subcore_axis_name="s", num_cores=NC, num_subcores=NS
  )


_SC_PARAMS = pltpu.CompilerParams(needs_layout_passes=False)
_SC_PARAMS_LINEAR = pltpu.CompilerParams(
    needs_layout_passes=False, use_tc_tiling_on_sc=False
)


def _mm1_body(x_ref, w_ref, as_ref, ad_ref, *out_refs, ng, dg):
  h_refs = out_refs[:ng]
  asrc_ref, adst_ref, exs_ref = out_refs[ng:]
  h = jnp.dot(x_ref[...], w_ref[...], preferred_element_type=_f32)
  for gi in range(ng):
    h_refs[gi][...] = h[:, gi * dg:(gi + 1) * dg]
  asrc = jnp.sum(h * as_ref[...], axis=1, keepdims=True)
  adst = jnp.sum(h * ad_ref[...], axis=1, keepdims=True)
  asrc_ref[...] = asrc
  adst_ref[...] = adst
  al = asrc + adst
  al = jnp.where(al >= 0.0, al, 0.2 * al)
  exs_ref[...] = jnp.exp(al)


def _mm_first(x, w, a_s, a_d, ng):
  r = 1000
  grid = (NN // r,)
  din = x.shape[1]
  dout = w.shape[1]
  dg = dout // ng
  outs = pl.pallas_call(
      functools.partial(_mm1_body, ng=ng, dg=dg),
      grid=grid,
      in_specs=[
          pl.BlockSpec((r, din), lambda i: (i, 0)),
          pl.BlockSpec((din, dout), lambda i: (0, 0)),
          pl.BlockSpec((1, dout), lambda i: (0, 0)),
          pl.BlockSpec((1, dout), lambda i: (0, 0)),
      ],
      out_specs=[pl.BlockSpec((r, dg), lambda i: (i, 0))] * ng
      + [pl.BlockSpec((r, 1), lambda i: (i, 0))] * 3,
      out_shape=[jax.ShapeDtypeStruct((NN, dg), _f32)] * ng
      + [jax.ShapeDtypeStruct((NN, 1), _f32)] * 3,
  )(x, w, a_s.reshape(1, dout), a_d.reshape(1, dout))
  return tuple(outs[:ng]), outs[ng], outs[ng + 1], outs[ng + 2]


def _mm2_body(*refs, npg, ng, dg, relu_out):
  op_ref = refs[0]
  hp_refs = refs[1:1 + npg]
  exs_ref, den_ref, b_ref, w_ref, as_ref, ad_ref = refs[1 + npg:7 + npg]
  out_refs = refs[7 + npg:]
  h_refs = out_refs[:ng]
  asrc_ref, adst_ref, exso_ref = out_refs[ng:]
  coef = exs_ref[...] / (den_ref[...] + 1e-16)
  hp = jnp.concatenate([hr[...] for hr in hp_refs], axis=1)
  xl = op_ref[0] + op_ref[1] + hp * coef + b_ref[...]
  if relu_out:
    xl = jnp.maximum(xl, 0.0)
  h = jnp.dot(xl, w_ref[...], preferred_element_type=_f32)
  for gi in range(ng):
    h_refs[gi][...] = h[:, gi * dg:(gi + 1) * dg]
  asrc = jnp.sum(h * as_ref[...], axis=1, keepdims=True)
  adst = jnp.sum(h * ad_ref[...], axis=1, keepdims=True)
  asrc_ref[...] = asrc
  adst_ref[...] = adst
  al = asrc + adst
  al = jnp.where(al >= 0.0, al, 0.2 * al)
  exso_ref[...] = jnp.exp(al)


def _mm_next(op, hp_groups, exs_prev, den_prev, b_prev, w, a_s, a_d, ng):
  r = 1000
  grid = (NN // r,)
  npg = len(hp_groups)
  dpg = hp_groups[0].shape[1]
  dp = npg * dpg
  dout = w.shape[1]
  dg = dout // ng
  outs = pl.pallas_call(
      functools.partial(_mm2_body, npg=npg, ng=ng, dg=dg, relu_out=True),
      grid=grid,
      in_specs=[pl.BlockSpec((2, r, dp), lambda i: (0, i, 0))]
      + [pl.BlockSpec((r, dpg), lambda i: (i, 0))] * npg
      + [
          pl.BlockSpec((r, 1), lambda i: (i, 0)),
          pl.BlockSpec((r, 1), lambda i: (i, 0)),
          pl.BlockSpec((1, dp), lambda i: (0, 0)),
          pl.BlockSpec((dp, dout), lambda i: (0, 0)),
          pl.BlockSpec((1, dout), lambda i: (0, 0)),
          pl.BlockSpec((1, dout), lambda i: (0, 0)),
      ],
      out_specs=[pl.BlockSpec((r, dg), lambda i: (i, 0))] * ng
      + [pl.BlockSpec((r, 1), lambda i: (i, 0))] * 3,
      out_shape=[jax.ShapeDtypeStruct((NN, dg), _f32)] * ng
      + [jax.ShapeDtypeStruct((NN, 1), _f32)] * 3,
  )(op, *hp_groups, exs_prev, den_prev, b_prev.reshape(1, dp), w,
    a_s.reshape(1, dout), a_d.reshape(1, dout))
  return tuple(outs[:ng]), outs[ng], outs[ng + 1], outs[ng + 2]


def _fin_body(l0_ref, l1_ref, h0_ref, h1_ref, exs_ref, den_ref, b_ref,
              out_ref):
  coef = exs_ref[...] / (den_ref[...] + 1e-16)
  l0 = l0_ref[...] + h0_ref[...] * coef + b_ref[0, 0]
  l1 = l1_ref[...] + h1_ref[...] * coef + b_ref[0, 1]
  m = jnp.maximum(l0, l1)
  e0 = jnp.exp(l0 - m)
  e1 = jnp.exp(l1 - m)
  s = e0 + e1
  out_ref[...] = jnp.concatenate([e0 / s, e1 / s], axis=1)


def _fin(l0, l1, h0, h1, exs_prev, den_prev, b_prev):
  r = 1000
  grid = (NN // r,)
  return pl.pallas_call(
      _fin_body,
      grid=grid,
      in_specs=[pl.BlockSpec((r, 1), lambda i: (i, 0))] * 6
      + [pl.BlockSpec((1, 2), lambda i: (0, 0))],
      out_specs=pl.BlockSpec((r, 2), lambda i: (i, 0)),
      out_shape=jax.ShapeDtypeStruct((NN, 2), _f32),
  )(l0, l1, h0, h1, exs_prev, den_prev, b_prev.reshape(1, 2))


def _edge_ex_body(src_hbm, dst_hbm, as_hbm, ad_hbm, ex_hbm, dp_hbm,
                  asv, adv, srcv, dstv, exv, zv, dacc):
  c = lax.axis_index("c")
  s = lax.axis_index("s")
  wid = c * NS + s
  pltpu.sync_copy(as_hbm, asv)
  pltpu.sync_copy(ad_hbm, adv)

  zeros16 = jnp.zeros((16,), _f32)

  def _zloop(i, _):
    zv[pl.ds(i * 16, 16)] = zeros16
    return 0

  lax.fori_loop(0, SUBSEG // 16, _zloop, 0)
  pltpu.sync_copy(zv, dacc.at[pl.ds(s * SUBSEG, SUBSEG)])
  plsc.subcore_barrier()

  iota16 = lax.broadcasted_iota(_i32, (16,), 0)

  def _slab(t, _):
    off = (wid * NSLAB + t) * SLAB
    pltpu.sync_copy(src_hbm.at[pl.ds(off, SLAB)], srcv)
    for i in range(12):
      pltpu.sync_copy(dst_hbm.at[pl.ds(off + i * G, G)], dstv.at[i])

    def _vec(j, _):
      r = j // 8
      cc = (j % 8) * 16
      d = dstv[r, pl.ds(cc, 16)]
      si = srcv[pl.ds(j * 16, 16)]
      av = plsc.load_gather(asv, [si])
      bv = plsc.load_gather(adv, [d])
      al = av + bv
      al = jnp.where(al >= 0.0, al, 0.2 * al)
      ex = jnp.exp(al)
      gidx = off + j * 16 + iota16
      ex = jnp.where(gidx < EE, ex, 0.0)
      exv[pl.ds(j * 16, 16)] = ex
      return 0

    lax.fori_loop(0, SLAB // 16, _vec, 0)
    pltpu.sync_copy(exv, ex_hbm.at[pl.ds(off, SLAB)])
    for i in range(12):
      pltpu.sync_copy(exv.at[pl.ds(i * G, G)], dacc.at[dstv.at[i]], add=True)
    return 0

  lax.fori_loop(0, NSLAB, _slab, 0)
  plsc.subcore_barrier()
  pltpu.sync_copy(dacc.at[pl.ds(s * SUBSEG, SUBSEG)],
                  dp_hbm.at[pl.ds(c * NPD + s * SUBSEG, SUBSEG)])


def _edge_ex(srcp, dstp, asrc, adst):
  k = pl.kernel(
      _edge_ex_body,
      out_type=[
          jax.ShapeDtypeStruct((EP,), _f32),
          jax.ShapeDtypeStruct((NC * NPD,), _f32),
      ],
      mesh=_mesh(),
      compiler_params=_SC_PARAMS,
      scratch_types=[
          pltpu.VMEM((NN,), _f32),
          pltpu.VMEM((NN,), _f32),
          pltpu.VMEM((SLAB,), _i32),
          pltpu.VMEM((12, G), _i32),
          pltpu.VMEM((SLAB,), _f32),
          pltpu.VMEM((SUBSEG,), _f32),
          pltpu.VMEM_SHARED((NPD,), _f32),
      ],
  )
  return k(srcp, dstp, asrc, adst)


def _coef_body(dst_hbm, ex_hbm, dp_hbm, exs_hbm, coef_hbm, den_hbm,
               dnv, tmpv, dstv, exv, cfv):
  c = lax.axis_index("c")
  s = lax.axis_index("s")
  wid = c * NS + s

  pltpu.sync_copy(dp_hbm.at[pl.ds(0, NPD)], dnv)
  pltpu.sync_copy(dp_hbm.at[pl.ds(NPD, NPD)], tmpv)

  def _add1(j, _):
    sl = pl.ds(j * 16, 16)
    dnv[sl] = dnv[sl] + tmpv[sl]
    return 0

  lax.fori_loop(0, NPD // 16, _add1, 0)
  pltpu.sync_copy(exs_hbm, tmpv)
  lax.fori_loop(0, NPD // 16, _add1, 0)

  @pl.when(wid == 0)
  def _():
    pltpu.sync_copy(dnv, den_hbm)

  def _slab(t, _):
    off = (wid * NSLAB + t) * SLAB
    pltpu.sync_copy(dst_hbm.at[pl.ds(off, SLAB)], dstv)
    pltpu.sync_copy(ex_hbm.at[pl.ds(off, SLAB)], exv)

    def _vec(j, _):
      sl = pl.ds(j * 16, 16)
      d = dstv[sl]
      ex = exv[sl]
      den = plsc.load_gather(dnv, [d])
      cfv[sl] = ex / (den + 1e-16)
      return 0

    lax.fori_loop(0, SLAB // 16, _vec, 0)
    pltpu.sync_copy(cfv, coef_hbm.at[pl.ds(off, SLAB)])
    return 0

  lax.fori_loop(0, NSLAB, _slab, 0)


def _coef(dstp, expe, dpart, exself):
  k = pl.kernel(
      _coef_body,
      out_type=[
          jax.ShapeDtypeStruct((EP,), _f32),
          jax.ShapeDtypeStruct((NPD,), _f32),
      ],
      mesh=_mesh(),
      compiler_params=_SC_PARAMS,
      scratch_types=[
          pltpu.VMEM((NPD,), _f32),
          pltpu.VMEM((NPD,), _f32),
          pltpu.VMEM((SLAB,), _i32),
          pltpu.VMEM((SLAB,), _f32),
          pltpu.VMEM((SLAB,), _f32),
      ],
  )
  return k(dstp, expe, dpart, exself)


def _agg_body(*refs, ng, dg, bn, nb):
  src_hbm, dst_hbm, cf_hbm = refs[:3]
  h_hbms = refs[3:3 + ng]
  op_hbm = refs[3 + ng]
  srcv, dstv, cfv, src_c, dl_c, cf_c, dl2d = refs[4 + ng:11 + ng]
  rows_gs = refs[11 + ng:11 + 2 * ng]
  zrow = refs[11 + 2 * ng]
  accs = refs[12 + 2 * ng:12 + 3 * ng]

  c = lax.axis_index("c")
  s = lax.axis_index("s")
  wid = c * NS + s
  dt = dg // 16
  rps = bn // NS
  dtot = ng * dg

  zeros16 = jnp.zeros((16,), _f32)
  izeros16 = jnp.zeros((16,), _i32)

  def _zrow_init(r, _):
    for t2 in range(dt):
      zrow[r, pl.ds(t2 * 16, 16)] = zeros16
    return 0

  lax.fori_loop(0, 16, _zrow_init, 0)

  def _bucket(b, _):
    lo = b * bn

    def _zacc(z, _):
      for gi in range(ng):
        pltpu.sync_copy(zrow, accs[gi].at[pl.ds(s * rps + z * 16, 16)])
      return 0

    lax.fori_loop(0, rps // 16, _zacc, 0)
    plsc.subcore_barrier()

    def _slab(t, _):
      off = (wid * NSLAB + t) * SLAB
      pltpu.sync_copy(src_hbm.at[pl.ds(off, SLAB)], srcv)
      pltpu.sync_copy(dst_hbm.at[pl.ds(off, SLAB)], dstv)
      pltpu.sync_copy(cf_hbm.at[pl.ds(off, SLAB)], cfv)

      def _vec(j, cnt):
        sl = pl.ds(j * 16, 16)
        dd = dstv[sl]
        within = (dd >= lo) & (dd < lo + bn)
        plsc.store_compressed(src_c.at[pl.ds(cnt, 16)], srcv[sl], mask=within)
        plsc.store_compressed(dl_c.at[pl.ds(cnt, 16)], dd - lo, mask=within)
        plsc.store_compressed(cf_c.at[pl.ds(cnt, 16)], cfv[sl], mask=within)
        pc = plsc.all_reduce_population_count(within)
        return cnt + pc[0]

      cnt = lax.fori_loop(0, SLAB // 16, _vec, jnp.int32(0))
      for z in range(8):
        sl = pl.ds(cnt + z * 16, 16)
        src_c[sl] = izeros16
        dl_c[sl] = izeros16
        cf_c[sl] = zeros16
      nch = (cnt + G - 1) // G

      def _chunk(i, _):
        for k in range(G // 16):
          dl2d[0, pl.ds(k * 16, 16)] = dl_c[pl.ds(i * G + k * 16, 16)]
        for gi in range(ng):
          pltpu.sync_copy(h_hbms[gi].at[src_c.at[pl.ds(i * G, G)]],
                          rows_gs[gi])

        def _wrow(r, _):
          base = jnp.zeros((16,), _i32) + (i * G + r)
          cv = plsc.load_gather(cf_c, [base])
          for gi in range(ng):
            for t2 in range(dt):
              sl2 = pl.ds(t2 * 16, 16)
              rows_gs[gi][r, sl2] = rows_gs[gi][r, sl2] * cv
          return 0

        lax.fori_loop(0, G, _wrow, 0)
        for gi in range(ng):
          pltpu.sync_copy(rows_gs[gi], accs[gi].at[dl2d.at[0]], add=True)
        return 0

      lax.fori_loop(0, nch, _chunk, 0)
      return 0

    lax.fori_loop(0, NSLAB, _slab, 0)
    plsc.subcore_barrier()
    for gi in range(ng):
      if ng == 1:
        dst_slc = op_hbm.at[c, pl.ds(lo + s * rps, rps)]
      else:
        dst_slc = op_hbm.at[c, pl.ds(lo + s * rps, rps), pl.ds(gi * dg, dg)]
      pltpu.sync_copy(accs[gi].at[pl.ds(s * rps, rps)], dst_slc)
    return 0

  lax.fori_loop(0, nb, _bucket, 0)


def _agg(srcp, dstp, coefe, h_groups, bn, nb):
  ng = len(h_groups)
  dg = h_groups[0].shape[1]
  od = bn * nb
  body = functools.partial(_agg_body, ng=ng, dg=dg, bn=bn, nb=nb)
  k = pl.kernel(
      body,
      out_type=jax.ShapeDtypeStruct((NC, od, ng * dg), _f32),
      mesh=_mesh(),
      compiler_params=_SC_PARAMS_LINEAR,
      scratch_types=[
          pltpu.VMEM((SLAB,), _i32),
          pltpu.VMEM((SLAB,), _i32),
          pltpu.VMEM((SLAB,), _f32),
          pltpu.VMEM((SLAB + G,), _i32),
          pltpu.VMEM((SLAB + G,), _i32),
          pltpu.VMEM((SLAB + G,), _f32),
          pltpu.VMEM((1, G), _i32),
      ]
      + [pltpu.VMEM((G, dg), _f32)] * ng
      + [pltpu.VMEM((16, dg), _f32)]
      + [pltpu.VMEM_SHARED((bn, dg), _f32)] * ng,
  )
  return k(srcp, dstp, coefe, *h_groups)


def _agg_pair_body(src_hbm, dst_hbm, cf_hbm, h0_hbm, h1_hbm, p0_hbm, p1_hbm,
                   h0v, h1v, srcv, dstv, cfv, v0v, v1v, zv, acc0, acc1):
  c = lax.axis_index("c")
  s = lax.axis_index("s")
  wid = c * NS + s
  pltpu.sync_copy(h0_hbm, h0v)
  pltpu.sync_copy(h1_hbm, h1v)

  zeros16 = jnp.zeros((16,), _f32)

  def _zloop(i, _):
    zv[pl.ds(i * 16, 16)] = zeros16
    return 0

  lax.fori_loop(0, SUBSEG // 16, _zloop, 0)
  pltpu.sync_copy(zv, acc0.at[pl.ds(s * SUBSEG, SUBSEG)])
  pltpu.sync_copy(zv, acc1.at[pl.ds(s * SUBSEG, SUBSEG)])
  plsc.subcore_barrier()

  def _slab(t, _):
    off = (wid * NSLAB + t) * SLAB
    pltpu.sync_copy(src_hbm.at[pl.ds(off, SLAB)], srcv)
    pltpu.sync_copy(cf_hbm.at[pl.ds(off, SLAB)], cfv)
    for i in range(12):
      pltpu.sync_copy(dst_hbm.at[pl.ds(off + i * G, G)], dstv.at[i])

    def _vec(j, _):
      sl = pl.ds(j * 16, 16)
      si = srcv[sl]
      cf = cfv[sl]
      v0v[sl] = plsc.load_gather(h0v, [si]) * cf
      v1v[sl] = plsc.load_gather(h1v, [si]) * cf
      return 0

    lax.fori_loop(0, SLAB // 16, _vec, 0)
    for i in range(12):
      pltpu.sync_copy(v0v.at[pl.ds(i * G, G)], acc0.at[dstv.at[i]], add=True)
      pltpu.sync_copy(v1v.at[pl.ds(i * G, G)], acc1.at[dstv.at[i]], add=True)
    return 0

  lax.fori_loop(0, NSLAB, _slab, 0)
  plsc.subcore_barrier()
  pltpu.sync_copy(acc0.at[pl.ds(s * SUBSEG, SUBSEG)],
                  p0_hbm.at[pl.ds(c * NPD + s * SUBSEG, SUBSEG)])
  pltpu.sync_copy(acc1.at[pl.ds(s * SUBSEG, SUBSEG)],
                  p1_hbm.at[pl.ds(c * NPD + s * SUBSEG, SUBSEG)])


def _agg_pair(srcp, dstp, coefe, h0, h1):
  k = pl.kernel(
      _agg_pair_body,
      out_type=[
          jax.ShapeDtypeStruct((NC * NPD,), _f32),
          jax.ShapeDtypeStruct((NC * NPD,), _f32),
      ],
      mesh=_mesh(),
      compiler_params=_SC_PARAMS,
      scratch_types=[
          pltpu.VMEM((NN,), _f32),
          pltpu.VMEM((NN,), _f32),
          pltpu.VMEM((SLAB,), _i32),
          pltpu.VMEM((12, G), _i32),
          pltpu.VMEM((SLAB,), _f32),
          pltpu.VMEM((SLAB,), _f32),
          pltpu.VMEM((SLAB,), _f32),
          pltpu.VMEM((SUBSEG,), _f32),
          pltpu.VMEM_SHARED((NPD,), _f32),
          pltpu.VMEM_SHARED((NPD,), _f32),
      ],
  )
  return k(srcp, dstp, coefe, h0, h1)


def _layer_edges(srcp, dstp, h_groups, asrc, adst, exself, bn, nb):
  expe, dpart = _edge_ex(srcp, dstp, asrc.reshape(-1), adst.reshape(-1))
  exsp = jnp.concatenate([exself.reshape(-1), jnp.zeros((NPD - NN,), _f32)])
  coefe, denom = _coef(dstp, expe, dpart, exsp)
  opart = _agg(srcp, dstp, coefe, h_groups, bn, nb)
  return opart, denom[:NN].reshape(NN, 1)


def kernel(x, edge_index, batch, W1, a_s1, a_d1, b1, W2, a_s2, a_d2, b2,
           W3, a_s3, a_d3, b3):
  del batch
  pad = jnp.zeros((EP - EE,), dtype=edge_index.dtype)
  srcp = jnp.concatenate([edge_index[0], pad])
  dstp = jnp.concatenate([edge_index[1], pad])

  h1g, asrc1, adst1, exs1 = _mm_first(x, W1, a_s1, a_d1, ng=1)
  op1, den1 = _layer_edges(srcp, dstp, h1g, asrc1, adst1, exs1, 8192, 7)

  h2g, asrc2, adst2, exs2 = _mm_next(op1, h1g, exs1, den1, b1, W2, a_s2,
                                     a_d2, ng=2)
  op2, den2 = _layer_edges(srcp, dstp, h2g, asrc2, adst2, exs2, 4096, 13)

  w3p = jnp.concatenate([W3, jnp.zeros((W3.shape[0], 14), _f32)], axis=1)
  as3p = jnp.concatenate([a_s3, jnp.zeros((14,), _f32)])
  ad3p = jnp.concatenate([a_d3, jnp.zeros((14,), _f32)])
  b3p = jnp.concatenate([b3, jnp.zeros((14,), _f32)])
  h3g, asrc3, adst3, exs3 = _mm_next(op2, h2g, exs2, den2, b2, w3p, as3p,
                                     ad3p, ng=1)
  h3 = h3g[0]
  expe3, dpart3 = _edge_ex(srcp, dstp, asrc3.reshape(-1), adst3.reshape(-1))
  exsp3 = jnp.concatenate([exs3.reshape(-1), jnp.zeros((NPD - NN,), _f32)])
  coef3, den3 = _coef(dstp, expe3, dpart3, exsp3)
  h3c0 = h3[:, 0]
  h3c1 = h3[:, 1]
  p0, p1 = _agg_pair(srcp, dstp, coef3, h3c0, h3c1)
  l0 = (p0[:NPD] + p0[NPD:])[:NN].reshape(NN, 1)
  l1 = (p1[:NPD] + p1[NPD:])[:NN].reshape(NN, 1)

  return _fin(l0, l1, h3c0.reshape(NN, 1), h3c1.reshape(NN, 1), exs3,
              den3[:NN].reshape(NN, 1), b3)


# speedup vs baseline: 1.0759x; 1.0759x over previous
import functools

import jax
import jax.numpy as jnp
from jax import lax
from jax.experimental import pallas as pl
from jax.experimental.pallas import tpu as pltpu
from jax.experimental.pallas import tpu_sc as plsc

NN = 50000
EE = 800000
NC = 2
NS = 16
NW = NC * NS
SLAB = 1536
NSLAB = 17
CH = SLAB * NSLAB
EP = CH * NW
NPD = 53248
SUBSEG = NPD // NS
G = 128

_f32 = jnp.float32
_i32 = jnp.int32


def _mesh():
  return plsc.VectorSubcoreMesh(
      core_axis_name="c", subcore_axis_name="s", num_cores=NC, num_subcores=NS
  )


_SC_PARAMS = pltpu.CompilerParams(needs_layout_passes=False)
_SC_PARAMS_LINEAR = pltpu.CompilerParams(
    needs_layout_passes=False, use_tc_tiling_on_sc=False
)


def _mm1_body(x_ref, w_ref, as_ref, ad_ref, *out_refs, ng, dg):
  h_refs = out_refs[:ng]
  asrc_ref, adst_ref, exs_ref = out_refs[ng:]
  h = jnp.dot(x_ref[...], w_ref[...], preferred_element_type=_f32)
  for gi in range(ng):
    h_refs[gi][...] = h[:, gi * dg:(gi + 1) * dg]
  asrc = jnp.sum(h * as_ref[...], axis=1, keepdims=True)
  adst = jnp.sum(h * ad_ref[...], axis=1, keepdims=True)
  asrc_ref[...] = asrc
  adst_ref[...] = adst
  al = asrc + adst
  al = jnp.where(al >= 0.0, al, 0.2 * al)
  exs_ref[...] = jnp.exp(al)


def _mm_first(x, w, a_s, a_d, ng):
  r = 1000
  grid = (NN // r,)
  din = x.shape[1]
  dout = w.shape[1]
  dg = dout // ng
  outs = pl.pallas_call(
      functools.partial(_mm1_body, ng=ng, dg=dg),
      grid=grid,
      in_specs=[
          pl.BlockSpec((r, din), lambda i: (i, 0)),
          pl.BlockSpec((din, dout), lambda i: (0, 0)),
          pl.BlockSpec((1, dout), lambda i: (0, 0)),
          pl.BlockSpec((1, dout), lambda i: (0, 0)),
      ],
      out_specs=[pl.BlockSpec((r, dg), lambda i: (i, 0))] * ng
      + [pl.BlockSpec((r, 1), lambda i: (i, 0))] * 3,
      out_shape=[jax.ShapeDtypeStruct((NN, dg), _f32)] * ng
      + [jax.ShapeDtypeStruct((NN, 1), _f32)] * 3,
  )(x, w, a_s.reshape(1, dout), a_d.reshape(1, dout))
  return tuple(outs[:ng]), outs[ng], outs[ng + 1], outs[ng + 2]


def _mm2_body(*refs, npg, ng, dg, relu_out):
  op_ref = refs[0]
  hp_refs = refs[1:1 + npg]
  exs_ref, den_ref, b_ref, w_ref, as_ref, ad_ref = refs[1 + npg:7 + npg]
  out_refs = refs[7 + npg:]
  h_refs = out_refs[:ng]
  asrc_ref, adst_ref, exso_ref = out_refs[ng:]
  coef = exs_ref[...] / (den_ref[...] + 1e-16)
  hp = jnp.concatenate([hr[...] for hr in hp_refs], axis=1)
  xl = op_ref[0] + op_ref[1] + hp * coef + b_ref[...]
  if relu_out:
    xl = jnp.maximum(xl, 0.0)
  h = jnp.dot(xl, w_ref[...], preferred_element_type=_f32)
  for gi in range(ng):
    h_refs[gi][...] = h[:, gi * dg:(gi + 1) * dg]
  asrc = jnp.sum(h * as_ref[...], axis=1, keepdims=True)
  adst = jnp.sum(h * ad_ref[...], axis=1, keepdims=True)
  asrc_ref[...] = asrc
  adst_ref[...] = adst
  al = asrc + adst
  al = jnp.where(al >= 0.0, al, 0.2 * al)
  exso_ref[...] = jnp.exp(al)


def _mm_next(op, hp_groups, exs_prev, den_prev, b_prev, w, a_s, a_d, ng):
  r = 1000
  grid = (NN // r,)
  npg = len(hp_groups)
  dpg = hp_groups[0].shape[1]
  dp = npg * dpg
  dout = w.shape[1]
  dg = dout // ng
  outs = pl.pallas_call(
      functools.partial(_mm2_body, npg=npg, ng=ng, dg=dg, relu_out=True),
      grid=grid,
      in_specs=[pl.BlockSpec((2, r, dp), lambda i: (0, i, 0))]
      + [pl.BlockSpec((r, dpg), lambda i: (i, 0))] * npg
      + [
          pl.BlockSpec((r, 1), lambda i: (i, 0)),
          pl.BlockSpec((r, 1), lambda i: (i, 0)),
          pl.BlockSpec((1, dp), lambda i: (0, 0)),
          pl.BlockSpec((dp, dout), lambda i: (0, 0)),
          pl.BlockSpec((1, dout), lambda i: (0, 0)),
          pl.BlockSpec((1, dout), lambda i: (0, 0)),
      ],
      out_specs=[pl.BlockSpec((r, dg), lambda i: (i, 0))] * ng
      + [pl.BlockSpec((r, 1), lambda i: (i, 0))] * 3,
      out_shape=[jax.ShapeDtypeStruct((NN, dg), _f32)] * ng
      + [jax.ShapeDtypeStruct((NN, 1), _f32)] * 3,
  )(op, *hp_groups, exs_prev, den_prev, b_prev.reshape(1, dp), w,
    a_s.reshape(1, dout), a_d.reshape(1, dout))
  return tuple(outs[:ng]), outs[ng], outs[ng + 1], outs[ng + 2]


def _fin_body(l0_ref, l1_ref, h0_ref, h1_ref, exs_ref, den_ref, b_ref,
              out_ref):
  coef = exs_ref[...] / (den_ref[...] + 1e-16)
  l0 = l0_ref[...] + h0_ref[...] * coef + b_ref[0, 0]
  l1 = l1_ref[...] + h1_ref[...] * coef + b_ref[0, 1]
  m = jnp.maximum(l0, l1)
  e0 = jnp.exp(l0 - m)
  e1 = jnp.exp(l1 - m)
  s = e0 + e1
  out_ref[...] = jnp.concatenate([e0 / s, e1 / s], axis=1)


def _fin(l0, l1, h0, h1, exs_prev, den_prev, b_prev):
  r = 1000
  grid = (NN // r,)
  return pl.pallas_call(
      _fin_body,
      grid=grid,
      in_specs=[pl.BlockSpec((r, 1), lambda i: (i, 0))] * 6
      + [pl.BlockSpec((1, 2), lambda i: (0, 0))],
      out_specs=pl.BlockSpec((r, 2), lambda i: (i, 0)),
      out_shape=jax.ShapeDtypeStruct((NN, 2), _f32),
  )(l0, l1, h0, h1, exs_prev, den_prev, b_prev.reshape(1, 2))


def _edge_ex_body(src_hbm, dst_hbm, as_hbm, ad_hbm, ex_hbm, dp_hbm,
                  asv, adv, srcv, dstv, exv, zv, dacc):
  c = lax.axis_index("c")
  s = lax.axis_index("s")
  wid = c * NS + s
  pltpu.sync_copy(as_hbm, asv)
  pltpu.sync_copy(ad_hbm, adv)

  zeros16 = jnp.zeros((16,), _f32)

  def _zloop(i, _):
    zv[pl.ds(i * 16, 16)] = zeros16
    return 0

  lax.fori_loop(0, SUBSEG // 16, _zloop, 0)
  pltpu.sync_copy(zv, dacc.at[pl.ds(s * SUBSEG, SUBSEG)])
  plsc.subcore_barrier()

  iota16 = lax.broadcasted_iota(_i32, (16,), 0)

  def _slab(t, _):
    off = (wid * NSLAB + t) * SLAB
    pltpu.sync_copy(src_hbm.at[pl.ds(off, SLAB)], srcv)
    for i in range(12):
      pltpu.sync_copy(dst_hbm.at[pl.ds(off + i * G, G)], dstv.at[i])

    def _vec(j, _):
      r = j // 8
      cc = (j % 8) * 16
      d = dstv[r, pl.ds(cc, 16)]
      si = srcv[pl.ds(j * 16, 16)]
      av = plsc.load_gather(asv, [si])
      bv = plsc.load_gather(adv, [d])
      al = av + bv
      al = jnp.where(al >= 0.0, al, 0.2 * al)
      ex = jnp.exp(al)
      gidx = off + j * 16 + iota16
      ex = jnp.where(gidx < EE, ex, 0.0)
      exv[pl.ds(j * 16, 16)] = ex
      return 0

    lax.fori_loop(0, SLAB // 16, _vec, 0)
    pltpu.sync_copy(exv, ex_hbm.at[pl.ds(off, SLAB)])
    for i in range(12):
      pltpu.sync_copy(exv.at[pl.ds(i * G, G)], dacc.at[dstv.at[i]], add=True)
    return 0

  lax.fori_loop(0, NSLAB, _slab, 0)
  plsc.subcore_barrier()
  pltpu.sync_copy(dacc.at[pl.ds(s * SUBSEG, SUBSEG)],
                  dp_hbm.at[pl.ds(c * NPD + s * SUBSEG, SUBSEG)])


def _edge_ex(srcp, dstp, asrc, adst):
  k = pl.kernel(
      _edge_ex_body,
      out_type=[
          jax.ShapeDtypeStruct((EP,), _f32),
          jax.ShapeDtypeStruct((NC * NPD,), _f32),
      ],
      mesh=_mesh(),
      compiler_params=_SC_PARAMS,
      scratch_types=[
          pltpu.VMEM((NN,), _f32),
          pltpu.VMEM((NN,), _f32),
          pltpu.VMEM((SLAB,), _i32),
          pltpu.VMEM((12, G), _i32),
          pltpu.VMEM((SLAB,), _f32),
          pltpu.VMEM((SUBSEG,), _f32),
          pltpu.VMEM_SHARED((NPD,), _f32),
      ],
  )
  return k(srcp, dstp, asrc, adst)


def _coef_body(dst_hbm, ex_hbm, dp_hbm, exs_hbm, coef_hbm, den_hbm,
               dnv, tmpv, dstv, exv, cfv):
  c = lax.axis_index("c")
  s = lax.axis_index("s")
  wid = c * NS + s

  pltpu.sync_copy(dp_hbm.at[pl.ds(0, NPD)], dnv)
  pltpu.sync_copy(dp_hbm.at[pl.ds(NPD, NPD)], tmpv)

  def _add1(j, _):
    sl = pl.ds(j * 16, 16)
    dnv[sl] = dnv[sl] + tmpv[sl]
    return 0

  lax.fori_loop(0, NPD // 16, _add1, 0)
  pltpu.sync_copy(exs_hbm, tmpv)
  lax.fori_loop(0, NPD // 16, _add1, 0)

  @pl.when(wid == 0)
  def _():
    pltpu.sync_copy(dnv, den_hbm)

  def _slab(t, _):
    off = (wid * NSLAB + t) * SLAB
    pltpu.sync_copy(dst_hbm.at[pl.ds(off, SLAB)], dstv)
    pltpu.sync_copy(ex_hbm.at[pl.ds(off, SLAB)], exv)

    def _vec(j, _):
      sl = pl.ds(j * 16, 16)
      d = dstv[sl]
      ex = exv[sl]
      den = plsc.load_gather(dnv, [d])
      cfv[sl] = ex / (den + 1e-16)
      return 0

    lax.fori_loop(0, SLAB // 16, _vec, 0)
    pltpu.sync_copy(cfv, coef_hbm.at[pl.ds(off, SLAB)])
    return 0

  lax.fori_loop(0, NSLAB, _slab, 0)


def _coef(dstp, expe, dpart, exself):
  k = pl.kernel(
      _coef_body,
      out_type=[
          jax.ShapeDtypeStruct((EP,), _f32),
          jax.ShapeDtypeStruct((NPD,), _f32),
      ],
      mesh=_mesh(),
      compiler_params=_SC_PARAMS,
      scratch_types=[
          pltpu.VMEM((NPD,), _f32),
          pltpu.VMEM((NPD,), _f32),
          pltpu.VMEM((SLAB,), _i32),
          pltpu.VMEM((SLAB,), _f32),
          pltpu.VMEM((SLAB,), _f32),
      ],
  )
  return k(dstp, expe, dpart, exself)


def _agg_body(*refs, ng, dg, bn, nb):
  src_hbm, dst_hbm, cf_hbm = refs[:3]
  h_hbms = refs[3:3 + ng]
  op_hbm = refs[3 + ng]
  srcv, dstv, cfv, src_c, dl_c, cf_c, dl2d, sem = refs[4 + ng:12 + ng]
  rows_gs = refs[12 + ng:12 + 2 * ng]
  zrow = refs[12 + 2 * ng]
  accs = refs[13 + 2 * ng:13 + 3 * ng]

  c = lax.axis_index("c")
  s = lax.axis_index("s")
  wid = c * NS + s
  dt = dg // 16
  rps = bn // NS
  dtot = ng * dg

  zeros16 = jnp.zeros((16,), _f32)
  izeros16 = jnp.zeros((16,), _i32)

  def _zrow_init(r, _):
    for t2 in range(dt):
      zrow[r, pl.ds(t2 * 16, 16)] = zeros16
    return 0

  lax.fori_loop(0, 16, _zrow_init, 0)

  def _bucket(b, _):
    lo = b * bn

    def _zacc(z, _):
      for gi in range(ng):
        pltpu.sync_copy(zrow, accs[gi].at[pl.ds(s * rps + z * 16, 16)])
      return 0

    lax.fori_loop(0, rps // 16, _zacc, 0)
    plsc.subcore_barrier()

    def _slab(t, _):
      off = (wid * NSLAB + t) * SLAB
      pltpu.sync_copy(src_hbm.at[pl.ds(off, SLAB)], srcv)
      pltpu.sync_copy(dst_hbm.at[pl.ds(off, SLAB)], dstv)
      pltpu.sync_copy(cf_hbm.at[pl.ds(off, SLAB)], cfv)

      def _vec(j, cnt):
        sl = pl.ds(j * 16, 16)
        dd = dstv[sl]
        within = (dd >= lo) & (dd < lo + bn)
        plsc.store_compressed(src_c.at[pl.ds(cnt, 16)], srcv[sl], mask=within)
        plsc.store_compressed(dl_c.at[pl.ds(cnt, 16)], dd - lo, mask=within)
        plsc.store_compressed(cf_c.at[pl.ds(cnt, 16)], cfv[sl], mask=within)
        pc = plsc.all_reduce_population_count(within)
        return cnt + pc[0]

      cnt = lax.fori_loop(0, SLAB // 16, _vec, jnp.int32(0))
      for z in range(8):
        sl = pl.ds(cnt + z * 16, 16)
        src_c[sl] = izeros16
        dl_c[sl] = izeros16
        cf_c[sl] = zeros16
      nch = (cnt + G - 1) // G

      def _chunk(i, _):
        for k in range(G // 16):
          dl2d[0, pl.ds(k * 16, 16)] = dl_c[pl.ds(i * G + k * 16, 16)]
        def _start(k, _):
          sv = src_c[pl.ds(i * G + k * 16, 16)]
          for j in range(16):
            si = sv[j]
            for gi in range(ng):
              pltpu.async_copy(h_hbms[gi].at[si], rows_gs[gi].at[k * 16 + j],
                               sem)
          return 0

        lax.fori_loop(0, G // 16, _start, 0)
        for gi in range(ng):
          pltpu.make_async_copy(h_hbms[gi].at[pl.ds(0, G)], rows_gs[gi],
                                sem).wait()

        def _wrow(r, _):
          base = jnp.zeros((16,), _i32) + (i * G + r)
          cv = plsc.load_gather(cf_c, [base])
          for gi in range(ng):
            for t2 in range(dt):
              sl2 = pl.ds(t2 * 16, 16)
              rows_gs[gi][r, sl2] = rows_gs[gi][r, sl2] * cv
          return 0

        lax.fori_loop(0, G, _wrow, 0)
        for gi in range(ng):
          pltpu.sync_copy(rows_gs[gi], accs[gi].at[dl2d.at[0]], add=True)
        return 0

      lax.fori_loop(0, nch, _chunk, 0)
      return 0

    lax.fori_loop(0, NSLAB, _slab, 0)
    plsc.subcore_barrier()
    for gi in range(ng):
      if ng == 1:
        dst_slc = op_hbm.at[c, pl.ds(lo + s * rps, rps)]
      else:
        dst_slc = op_hbm.at[c, pl.ds(lo + s * rps, rps), pl.ds(gi * dg, dg)]
      pltpu.sync_copy(accs[gi].at[pl.ds(s * rps, rps)], dst_slc)
    return 0

  lax.fori_loop(0, nb, _bucket, 0)


def _agg(srcp, dstp, coefe, h_groups, bn, nb):
  ng = len(h_groups)
  dg = h_groups[0].shape[1]
  od = bn * nb
  body = functools.partial(_agg_body, ng=ng, dg=dg, bn=bn, nb=nb)
  k = pl.kernel(
      body,
      out_type=jax.ShapeDtypeStruct((NC, od, ng * dg), _f32),
      mesh=_mesh(),
      compiler_params=_SC_PARAMS_LINEAR,
      scratch_types=[
          pltpu.VMEM((SLAB,), _i32),
          pltpu.VMEM((SLAB,), _i32),
          pltpu.VMEM((SLAB,), _f32),
          pltpu.VMEM((SLAB + G,), _i32),
          pltpu.VMEM((SLAB + G,), _i32),
          pltpu.VMEM((SLAB + G,), _f32),
          pltpu.VMEM((1, G), _i32),
          pltpu.SemaphoreType.DMA,
      ]
      + [pltpu.VMEM((G, dg), _f32)] * ng
      + [pltpu.VMEM((16, dg), _f32)]
      + [pltpu.VMEM_SHARED((bn, dg), _f32)] * ng,
  )
  return k(srcp, dstp, coefe, *h_groups)


def _agg_pair_body(src_hbm, dst_hbm, cf_hbm, h0_hbm, h1_hbm, p0_hbm, p1_hbm,
                   h0v, h1v, srcv, dstv, cfv, v0v, v1v, zv, acc0, acc1):
  c = lax.axis_index("c")
  s = lax.axis_index("s")
  wid = c * NS + s
  pltpu.sync_copy(h0_hbm, h0v)
  pltpu.sync_copy(h1_hbm, h1v)

  zeros16 = jnp.zeros((16,), _f32)

  def _zloop(i, _):
    zv[pl.ds(i * 16, 16)] = zeros16
    return 0

  lax.fori_loop(0, SUBSEG // 16, _zloop, 0)
  pltpu.sync_copy(zv, acc0.at[pl.ds(s * SUBSEG, SUBSEG)])
  pltpu.sync_copy(zv, acc1.at[pl.ds(s * SUBSEG, SUBSEG)])
  plsc.subcore_barrier()

  def _slab(t, _):
    off = (wid * NSLAB + t) * SLAB
    pltpu.sync_copy(src_hbm.at[pl.ds(off, SLAB)], srcv)
    pltpu.sync_copy(cf_hbm.at[pl.ds(off, SLAB)], cfv)
    for i in range(12):
      pltpu.sync_copy(dst_hbm.at[pl.ds(off + i * G, G)], dstv.at[i])

    def _vec(j, _):
      sl = pl.ds(j * 16, 16)
      si = srcv[sl]
      cf = cfv[sl]
      v0v[sl] = plsc.load_gather(h0v, [si]) * cf
      v1v[sl] = plsc.load_gather(h1v, [si]) * cf
      return 0

    lax.fori_loop(0, SLAB // 16, _vec, 0)
    for i in range(12):
      pltpu.sync_copy(v0v.at[pl.ds(i * G, G)], acc0.at[dstv.at[i]], add=True)
      pltpu.sync_copy(v1v.at[pl.ds(i * G, G)], acc1.at[dstv.at[i]], add=True)
    return 0

  lax.fori_loop(0, NSLAB, _slab, 0)
  plsc.subcore_barrier()
  pltpu.sync_copy(acc0.at[pl.ds(s * SUBSEG, SUBSEG)],
                  p0_hbm.at[pl.ds(c * NPD + s * SUBSEG, SUBSEG)])
  pltpu.sync_copy(acc1.at[pl.ds(s * SUBSEG, SUBSEG)],
                  p1_hbm.at[pl.ds(c * NPD + s * SUBSEG, SUBSEG)])


def _agg_pair(srcp, dstp, coefe, h0, h1):
  k = pl.kernel(
      _agg_pair_body,
      out_type=[
          jax.ShapeDtypeStruct((NC * NPD,), _f32),
          jax.ShapeDtypeStruct((NC * NPD,), _f32),
      ],
      mesh=_mesh(),
      compiler_params=_SC_PARAMS,
      scratch_types=[
          pltpu.VMEM((NN,), _f32),
          pltpu.VMEM((NN,), _f32),
          pltpu.VMEM((SLAB,), _i32),
          pltpu.VMEM((12, G), _i32),
          pltpu.VMEM((SLAB,), _f32),
          pltpu.VMEM((SLAB,), _f32),
          pltpu.VMEM((SLAB,), _f32),
          pltpu.VMEM((SUBSEG,), _f32),
          pltpu.VMEM_SHARED((NPD,), _f32),
          pltpu.VMEM_SHARED((NPD,), _f32),
      ],
  )
  return k(srcp, dstp, coefe, h0, h1)


def _layer_edges(srcp, dstp, h_groups, asrc, adst, exself, bn, nb):
  expe, dpart = _edge_ex(srcp, dstp, asrc.reshape(-1), adst.reshape(-1))
  exsp = jnp.concatenate([exself.reshape(-1), jnp.zeros((NPD - NN,), _f32)])
  coefe, denom = _coef(dstp, expe, dpart, exsp)
  opart = _agg(srcp, dstp, coefe, h_groups, bn, nb)
  return opart, denom[:NN].reshape(NN, 1)


def kernel(x, edge_index, batch, W1, a_s1, a_d1, b1, W2, a_s2, a_d2, b2,
           W3, a_s3, a_d3, b3):
  del batch
  pad = jnp.zeros((EP - EE,), dtype=edge_index.dtype)
  srcp = jnp.concatenate([edge_index[0], pad])
  dstp = jnp.concatenate([edge_index[1], pad])

  h1g, asrc1, adst1, exs1 = _mm_first(x, W1, a_s1, a_d1, ng=1)
  op1, den1 = _layer_edges(srcp, dstp, h1g, asrc1, adst1, exs1, 8192, 7)

  h2g, asrc2, adst2, exs2 = _mm_next(op1, h1g, exs1, den1, b1, W2, a_s2,
                                     a_d2, ng=2)
  op2, den2 = _layer_edges(srcp, dstp, h2g, asrc2, adst2, exs2, 4096, 13)

  w3p = jnp.concatenate([W3, jnp.zeros((W3.shape[0], 14), _f32)], axis=1)
  as3p = jnp.concatenate([a_s3, jnp.zeros((14,), _f32)])
  ad3p = jnp.concatenate([a_d3, jnp.zeros((14,), _f32)])
  b3p = jnp.concatenate([b3, jnp.zeros((14,), _f32)])
  h3g, asrc3, adst3, exs3 = _mm_next(op2, h2g, exs2, den2, b2, w3p, as3p,
                                     ad3p, ng=1)
  h3 = h3g[0]
  expe3, dpart3 = _edge_ex(srcp, dstp, asrc3.reshape(-1), adst3.reshape(-1))
  exsp3 = jnp.concatenate([exs3.reshape(-1), jnp.zeros((NPD - NN,), _f32)])
  coef3, den3 = _coef(dstp, expe3, dpart3, exsp3)
  h3c0 = h3[:, 0]
  h3c1 = h3[:, 1]
  p0, p1 = _agg_pair(srcp, dstp, coef3, h3c0, h3c1)
  l0 = (p0[:NPD] + p0[NPD:])[:NN].reshape(NN, 1)
  l1 = (p1[:NPD] + p1[NPD:])[:NN].reshape(NN, 1)

  return _fin(l0, l1, h3c0.reshape(NN, 1), h3c1.reshape(NN, 1), exs3,
              den3[:NN].reshape(NN, 1), b3)


# speedup vs baseline: 1.0766x; 1.0006x over previous
import functools

import jax
import jax.numpy as jnp
from jax import lax
from jax.experimental import pallas as pl
from jax.experimental.pallas import tpu as pltpu
from jax.experimental.pallas import tpu_sc as plsc

NN = 50000
EE = 800000
NC = 2
NS = 16
NW = NC * NS
SLAB = 1536
NSLAB = 17
CH = SLAB * NSLAB
EP = CH * NW
NPD = 53248
SUBSEG = NPD // NS
G = 128

_f32 = jnp.float32
_i32 = jnp.int32


def _mesh():
  return plsc.VectorSubcoreMesh(
      core_axis_name="c", subcore_axis_name="s", num_cores=NC, num_subcores=NS
  )


_SC_PARAMS = pltpu.CompilerParams(needs_layout_passes=False)
_SC_PARAMS_LINEAR = pltpu.CompilerParams(
    needs_layout_passes=False, use_tc_tiling_on_sc=False
)


def _mm1_body(x_ref, w_ref, as_ref, ad_ref, *out_refs, ng, dg):
  h_refs = out_refs[:ng]
  asrc_ref, adst_ref, exs_ref = out_refs[ng:]
  h = jnp.dot(x_ref[...], w_ref[...], preferred_element_type=_f32)
  for gi in range(ng):
    h_refs[gi][...] = h[:, gi * dg:(gi + 1) * dg]
  asrc = jnp.sum(h * as_ref[...], axis=1, keepdims=True)
  adst = jnp.sum(h * ad_ref[...], axis=1, keepdims=True)
  asrc_ref[...] = asrc
  adst_ref[...] = adst
  al = asrc + adst
  al = jnp.where(al >= 0.0, al, 0.2 * al)
  exs_ref[...] = jnp.exp(al)


def _mm_first(x, w, a_s, a_d, ng):
  r = 1000
  grid = (NN // r,)
  din = x.shape[1]
  dout = w.shape[1]
  dg = dout // ng
  outs = pl.pallas_call(
      functools.partial(_mm1_body, ng=ng, dg=dg),
      grid=grid,
      in_specs=[
          pl.BlockSpec((r, din), lambda i: (i, 0)),
          pl.BlockSpec((din, dout), lambda i: (0, 0)),
          pl.BlockSpec((1, dout), lambda i: (0, 0)),
          pl.BlockSpec((1, dout), lambda i: (0, 0)),
      ],
      out_specs=[pl.BlockSpec((r, dg), lambda i: (i, 0))] * ng
      + [pl.BlockSpec((r, 1), lambda i: (i, 0))] * 3,
      out_shape=[jax.ShapeDtypeStruct((NN, dg), _f32)] * ng
      + [jax.ShapeDtypeStruct((NN, 1), _f32)] * 3,
  )(x, w, a_s.reshape(1, dout), a_d.reshape(1, dout))
  return tuple(outs[:ng]), outs[ng], outs[ng + 1], outs[ng + 2]


def _mm2_body(*refs, npg, ng, dg, relu_out):
  op_ref = refs[0]
  hp_refs = refs[1:1 + npg]
  exs_ref, den_ref, b_ref, w_ref, as_ref, ad_ref = refs[1 + npg:7 + npg]
  out_refs = refs[7 + npg:]
  h_refs = out_refs[:ng]
  asrc_ref, adst_ref, exso_ref = out_refs[ng:]
  coef = exs_ref[...] / (den_ref[...] + 1e-16)
  hp = jnp.concatenate([hr[...] for hr in hp_refs], axis=1)
  xl = op_ref[0] + op_ref[1] + hp * coef + b_ref[...]
  if relu_out:
    xl = jnp.maximum(xl, 0.0)
  h = jnp.dot(xl, w_ref[...], preferred_element_type=_f32)
  for gi in range(ng):
    h_refs[gi][...] = h[:, gi * dg:(gi + 1) * dg]
  asrc = jnp.sum(h * as_ref[...], axis=1, keepdims=True)
  adst = jnp.sum(h * ad_ref[...], axis=1, keepdims=True)
  asrc_ref[...] = asrc
  adst_ref[...] = adst
  al = asrc + adst
  al = jnp.where(al >= 0.0, al, 0.2 * al)
  exso_ref[...] = jnp.exp(al)


def _mm_next(op, hp_groups, exs_prev, den_prev, b_prev, w, a_s, a_d, ng):
  r = 1000
  grid = (NN // r,)
  npg = len(hp_groups)
  dpg = hp_groups[0].shape[1]
  dp = npg * dpg
  dout = w.shape[1]
  dg = dout // ng
  outs = pl.pallas_call(
      functools.partial(_mm2_body, npg=npg, ng=ng, dg=dg, relu_out=True),
      grid=grid,
      in_specs=[pl.BlockSpec((2, r, dp), lambda i: (0, i, 0))]
      + [pl.BlockSpec((r, dpg), lambda i: (i, 0))] * npg
      + [
          pl.BlockSpec((r, 1), lambda i: (i, 0)),
          pl.BlockSpec((r, 1), lambda i: (i, 0)),
          pl.BlockSpec((1, dp), lambda i: (0, 0)),
          pl.BlockSpec((dp, dout), lambda i: (0, 0)),
          pl.BlockSpec((1, dout), lambda i: (0, 0)),
          pl.BlockSpec((1, dout), lambda i: (0, 0)),
      ],
      out_specs=[pl.BlockSpec((r, dg), lambda i: (i, 0))] * ng
      + [pl.BlockSpec((r, 1), lambda i: (i, 0))] * 3,
      out_shape=[jax.ShapeDtypeStruct((NN, dg), _f32)] * ng
      + [jax.ShapeDtypeStruct((NN, 1), _f32)] * 3,
  )(op, *hp_groups, exs_prev, den_prev, b_prev.reshape(1, dp), w,
    a_s.reshape(1, dout), a_d.reshape(1, dout))
  return tuple(outs[:ng]), outs[ng], outs[ng + 1], outs[ng + 2]


def _fin_body(l0_ref, l1_ref, h0_ref, h1_ref, exs_ref, den_ref, b_ref,
              out_ref):
  coef = exs_ref[...] / (den_ref[...] + 1e-16)
  l0 = l0_ref[...] + h0_ref[...] * coef + b_ref[0, 0]
  l1 = l1_ref[...] + h1_ref[...] * coef + b_ref[0, 1]
  m = jnp.maximum(l0, l1)
  e0 = jnp.exp(l0 - m)
  e1 = jnp.exp(l1 - m)
  s = e0 + e1
  out_ref[...] = jnp.concatenate([e0 / s, e1 / s], axis=1)


def _fin(l0, l1, h0, h1, exs_prev, den_prev, b_prev):
  r = 1000
  grid = (NN // r,)
  return pl.pallas_call(
      _fin_body,
      grid=grid,
      in_specs=[pl.BlockSpec((r, 1), lambda i: (i, 0))] * 6
      + [pl.BlockSpec((1, 2), lambda i: (0, 0))],
      out_specs=pl.BlockSpec((r, 2), lambda i: (i, 0)),
      out_shape=jax.ShapeDtypeStruct((NN, 2), _f32),
  )(l0, l1, h0, h1, exs_prev, den_prev, b_prev.reshape(1, 2))


def _edge_ex_body(src_hbm, dst_hbm, as_hbm, ad_hbm, ex_hbm, dp_hbm,
                  asv, adv, srcv, dstv, exv, zv, dacc):
  c = lax.axis_index("c")
  s = lax.axis_index("s")
  wid = c * NS + s
  pltpu.sync_copy(as_hbm, asv)
  pltpu.sync_copy(ad_hbm, adv)

  zeros16 = jnp.zeros((16,), _f32)

  def _zloop(i, _):
    zv[pl.ds(i * 16, 16)] = zeros16
    return 0

  lax.fori_loop(0, SUBSEG // 16, _zloop, 0)
  pltpu.sync_copy(zv, dacc.at[pl.ds(s * SUBSEG, SUBSEG)])
  plsc.subcore_barrier()

  iota16 = lax.broadcasted_iota(_i32, (16,), 0)

  def _slab(t, _):
    off = (wid * NSLAB + t) * SLAB
    pltpu.sync_copy(src_hbm.at[pl.ds(off, SLAB)], srcv)
    for i in range(12):
      pltpu.sync_copy(dst_hbm.at[pl.ds(off + i * G, G)], dstv.at[i])

    def _vec(j, _):
      r = j // 8
      cc = (j % 8) * 16
      d = dstv[r, pl.ds(cc, 16)]
      si = srcv[pl.ds(j * 16, 16)]
      av = plsc.load_gather(asv, [si])
      bv = plsc.load_gather(adv, [d])
      al = av + bv
      al = jnp.where(al >= 0.0, al, 0.2 * al)
      ex = jnp.exp(al)
      gidx = off + j * 16 + iota16
      ex = jnp.where(gidx < EE, ex, 0.0)
      exv[pl.ds(j * 16, 16)] = ex
      return 0

    lax.fori_loop(0, SLAB // 16, _vec, 0)
    pltpu.sync_copy(exv, ex_hbm.at[pl.ds(off, SLAB)])
    for i in range(12):
      pltpu.sync_copy(exv.at[pl.ds(i * G, G)], dacc.at[dstv.at[i]], add=True)
    return 0

  lax.fori_loop(0, NSLAB, _slab, 0)
  plsc.subcore_barrier()
  pltpu.sync_copy(dacc.at[pl.ds(s * SUBSEG, SUBSEG)],
                  dp_hbm.at[pl.ds(c * NPD + s * SUBSEG, SUBSEG)])


def _edge_ex(srcp, dstp, asrc, adst):
  k = pl.kernel(
      _edge_ex_body,
      out_type=[
          jax.ShapeDtypeStruct((EP,), _f32),
          jax.ShapeDtypeStruct((NC * NPD,), _f32),
      ],
      mesh=_mesh(),
      compiler_params=_SC_PARAMS,
      scratch_types=[
          pltpu.VMEM((NN,), _f32),
          pltpu.VMEM((NN,), _f32),
          pltpu.VMEM((SLAB,), _i32),
          pltpu.VMEM((12, G), _i32),
          pltpu.VMEM((SLAB,), _f32),
          pltpu.VMEM((SUBSEG,), _f32),
          pltpu.VMEM_SHARED((NPD,), _f32),
      ],
  )
  return k(srcp, dstp, asrc, adst)


def _coef_body(dst_hbm, ex_hbm, dp_hbm, exs_hbm, coef_hbm, den_hbm,
               dnv, tmpv, dstv, exv, cfv):
  c = lax.axis_index("c")
  s = lax.axis_index("s")
  wid = c * NS + s

  pltpu.sync_copy(dp_hbm.at[pl.ds(0, NPD)], dnv)
  pltpu.sync_copy(dp_hbm.at[pl.ds(NPD, NPD)], tmpv)

  def _add1(j, _):
    sl = pl.ds(j * 16, 16)
    dnv[sl] = dnv[sl] + tmpv[sl]
    return 0

  lax.fori_loop(0, NPD // 16, _add1, 0)
  pltpu.sync_copy(exs_hbm, tmpv)
  lax.fori_loop(0, NPD // 16, _add1, 0)

  @pl.when(wid == 0)
  def _():
    pltpu.sync_copy(dnv, den_hbm)

  def _slab(t, _):
    off = (wid * NSLAB + t) * SLAB
    pltpu.sync_copy(dst_hbm.at[pl.ds(off, SLAB)], dstv)
    pltpu.sync_copy(ex_hbm.at[pl.ds(off, SLAB)], exv)

    def _vec(j, _):
      sl = pl.ds(j * 16, 16)
      d = dstv[sl]
      ex = exv[sl]
      den = plsc.load_gather(dnv, [d])
      cfv[sl] = ex / (den + 1e-16)
      return 0

    lax.fori_loop(0, SLAB // 16, _vec, 0)
    pltpu.sync_copy(cfv, coef_hbm.at[pl.ds(off, SLAB)])
    return 0

  lax.fori_loop(0, NSLAB, _slab, 0)


def _coef(dstp, expe, dpart, exself):
  k = pl.kernel(
      _coef_body,
      out_type=[
          jax.ShapeDtypeStruct((EP,), _f32),
          jax.ShapeDtypeStruct((NPD,), _f32),
      ],
      mesh=_mesh(),
      compiler_params=_SC_PARAMS,
      scratch_types=[
          pltpu.VMEM((NPD,), _f32),
          pltpu.VMEM((NPD,), _f32),
          pltpu.VMEM((SLAB,), _i32),
          pltpu.VMEM((SLAB,), _f32),
          pltpu.VMEM((SLAB,), _f32),
      ],
  )
  return k(dstp, expe, dpart, exself)


def _agg_body(*refs, ng, dg, bn, nb):
  src_hbm, dst_hbm, cf_hbm = refs[:3]
  h_hbms = refs[3:3 + ng]
  op_hbm = refs[3 + ng]
  srcv, dstv, cfv, src_c, dl_c, cf_c, dl2d, sem = refs[4 + ng:12 + ng]
  rows_gs = refs[12 + ng:12 + 2 * ng]
  zrow = refs[12 + 2 * ng]
  accs = refs[13 + 2 * ng:13 + 3 * ng]

  c = lax.axis_index("c")
  s = lax.axis_index("s")
  wid = c * NS + s
  dt = dg // 16
  rps = bn // NS

  zeros16 = jnp.zeros((16,), _f32)
  izeros16 = jnp.zeros((16,), _i32)

  def _zrow_init(r, _):
    for t2 in range(dt):
      zrow[r, pl.ds(t2 * 16, 16)] = zeros16
    return 0

  lax.fori_loop(0, 16, _zrow_init, 0)

  def _bucket(b, _):
    lo = b * bn

    def _zacc(z, _):
      for gi in range(ng):
        pltpu.sync_copy(zrow, accs[gi].at[pl.ds(s * rps + z * 16, 16)])
      return 0

    lax.fori_loop(0, rps // 16, _zacc, 0)
    plsc.subcore_barrier()

    def _slab(t, _):
      off = (wid * NSLAB + t) * SLAB
      pltpu.sync_copy(src_hbm.at[pl.ds(off, SLAB)], srcv)
      pltpu.sync_copy(dst_hbm.at[pl.ds(off, SLAB)], dstv)
      pltpu.sync_copy(cf_hbm.at[pl.ds(off, SLAB)], cfv)

      def _vec(j, cnt):
        sl = pl.ds(j * 16, 16)
        dd = dstv[sl]
        within = (dd >= lo) & (dd < lo + bn)
        plsc.store_compressed(src_c.at[pl.ds(cnt, 16)], srcv[sl], mask=within)
        plsc.store_compressed(dl_c.at[pl.ds(cnt, 16)], dd - lo, mask=within)
        plsc.store_compressed(cf_c.at[pl.ds(cnt, 16)], cfv[sl], mask=within)
        pc = plsc.all_reduce_population_count(within)
        return cnt + pc[0]

      cnt = lax.fori_loop(0, SLAB // 16, _vec, jnp.int32(0))
      for z in range(8):
        sl = pl.ds(cnt + z * 16, 16)
        src_c[sl] = izeros16
        dl_c[sl] = izeros16
        cf_c[sl] = zeros16
      nch = (cnt + G - 1) // G

      def _chunk(i, _):
        for k in range(G // 16):
          dl2d[0, pl.ds(k * 16, 16)] = dl_c[pl.ds(i * G + k * 16, 16)]
        def _start(k, _):
          sv = src_c[pl.ds(i * G + k * 16, 16)]
          for j in range(16):
            si = sv[j]
            for gi in range(ng):
              pltpu.async_copy(h_hbms[gi].at[si], rows_gs[gi].at[k * 16 + j],
                               sem)
          return 0

        lax.fori_loop(0, G // 16, _start, 0)
        for gi in range(ng):
          pltpu.make_async_copy(h_hbms[gi].at[pl.ds(0, G)], rows_gs[gi],
                                sem).wait()

        def _wrow(r, _):
          base = jnp.zeros((16,), _i32) + (i * G + r)
          cv = plsc.load_gather(cf_c, [base])
          for gi in range(ng):
            for t2 in range(dt):
              sl2 = pl.ds(t2 * 16, 16)
              rows_gs[gi][r, sl2] = rows_gs[gi][r, sl2] * cv
          return 0

        lax.fori_loop(0, G, _wrow, 0)
        for gi in range(ng):
          pltpu.sync_copy(rows_gs[gi], accs[gi].at[dl2d.at[0]], add=True)
        return 0

      lax.fori_loop(0, nch, _chunk, 0)
      return 0

    lax.fori_loop(0, NSLAB, _slab, 0)
    plsc.subcore_barrier()
    for gi in range(ng):
      if ng == 1:
        dst_slc = op_hbm.at[c, pl.ds(lo + s * rps, rps)]
      else:
        dst_slc = op_hbm.at[c, pl.ds(lo + s * rps, rps), pl.ds(gi * dg, dg)]
      pltpu.sync_copy(accs[gi].at[pl.ds(s * rps, rps)], dst_slc)
    return 0

  lax.fori_loop(0, nb, _bucket, 0)


def _agg(srcp, dstp, coefe, h_groups, bn, nb):
  ng = len(h_groups)
  dg = h_groups[0].shape[1]
  od = bn * nb
  body = functools.partial(_agg_body, ng=ng, dg=dg, bn=bn, nb=nb)
  k = pl.kernel(
      body,
      out_type=jax.ShapeDtypeStruct((NC, od, ng * dg), _f32),
      mesh=_mesh(),
      compiler_params=_SC_PARAMS_LINEAR,
      scratch_types=[
          pltpu.VMEM((SLAB,), _i32),
          pltpu.VMEM((SLAB,), _i32),
          pltpu.VMEM((SLAB,), _f32),
          pltpu.VMEM((SLAB + G,), _i32),
          pltpu.VMEM((SLAB + G,), _i32),
          pltpu.VMEM((SLAB + G,), _f32),
          pltpu.VMEM((1, G), _i32),
          pltpu.SemaphoreType.DMA,
      ]
      + [pltpu.VMEM((G, dg), _f32)] * ng
      + [pltpu.VMEM((16, dg), _f32)]
      + [pltpu.VMEM_SHARED((bn, dg), _f32)] * ng,
  )
  return k(srcp, dstp, coefe, *h_groups)


def _agg_pair_body(src_hbm, dst_hbm, cf_hbm, h0_hbm, h1_hbm, p0_hbm, p1_hbm,
                   h0v, h1v, srcv, dstv, cfv, v0v, v1v, zv, acc0, acc1):
  c = lax.axis_index("c")
  s = lax.axis_index("s")
  wid = c * NS + s
  pltpu.sync_copy(h0_hbm, h0v)
  pltpu.sync_copy(h1_hbm, h1v)

  zeros16 = jnp.zeros((16,), _f32)

  def _zloop(i, _):
    zv[pl.ds(i * 16, 16)] = zeros16
    return 0

  lax.fori_loop(0, SUBSEG // 16, _zloop, 0)
  pltpu.sync_copy(zv, acc0.at[pl.ds(s * SUBSEG, SUBSEG)])
  pltpu.sync_copy(zv, acc1.at[pl.ds(s * SUBSEG, SUBSEG)])
  plsc.subcore_barrier()

  def _slab(t, _):
    off = (wid * NSLAB + t) * SLAB
    pltpu.sync_copy(src_hbm.at[pl.ds(off, SLAB)], srcv)
    pltpu.sync_copy(cf_hbm.at[pl.ds(off, SLAB)], cfv)
    for i in range(12):
      pltpu.sync_copy(dst_hbm.at[pl.ds(off + i * G, G)], dstv.at[i])

    def _vec(j, _):
      sl = pl.ds(j * 16, 16)
      si = srcv[sl]
      cf = cfv[sl]
      v0v[sl] = plsc.load_gather(h0v, [si]) * cf
      v1v[sl] = plsc.load_gather(h1v, [si]) * cf
      return 0

    lax.fori_loop(0, SLAB // 16, _vec, 0)
    for i in range(12):
      pltpu.sync_copy(v0v.at[pl.ds(i * G, G)], acc0.at[dstv.at[i]], add=True)
      pltpu.sync_copy(v1v.at[pl.ds(i * G, G)], acc1.at[dstv.at[i]], add=True)
    return 0

  lax.fori_loop(0, NSLAB, _slab, 0)
  plsc.subcore_barrier()
  pltpu.sync_copy(acc0.at[pl.ds(s * SUBSEG, SUBSEG)],
                  p0_hbm.at[pl.ds(c * NPD + s * SUBSEG, SUBSEG)])
  pltpu.sync_copy(acc1.at[pl.ds(s * SUBSEG, SUBSEG)],
                  p1_hbm.at[pl.ds(c * NPD + s * SUBSEG, SUBSEG)])


def _agg_pair(srcp, dstp, coefe, h0, h1):
  k = pl.kernel(
      _agg_pair_body,
      out_type=[
          jax.ShapeDtypeStruct((NC * NPD,), _f32),
          jax.ShapeDtypeStruct((NC * NPD,), _f32),
      ],
      mesh=_mesh(),
      compiler_params=_SC_PARAMS,
      scratch_types=[
          pltpu.VMEM((NN,), _f32),
          pltpu.VMEM((NN,), _f32),
          pltpu.VMEM((SLAB,), _i32),
          pltpu.VMEM((12, G), _i32),
          pltpu.VMEM((SLAB,), _f32),
          pltpu.VMEM((SLAB,), _f32),
          pltpu.VMEM((SLAB,), _f32),
          pltpu.VMEM((SUBSEG,), _f32),
          pltpu.VMEM_SHARED((NPD,), _f32),
          pltpu.VMEM_SHARED((NPD,), _f32),
      ],
  )
  return k(srcp, dstp, coefe, h0, h1)


def _layer_edges(srcp, dstp, h_groups, asrc, adst, exself, bn, nb):
  expe, dpart = _edge_ex(srcp, dstp, asrc.reshape(-1), adst.reshape(-1))
  exsp = jnp.concatenate([exself.reshape(-1), jnp.zeros((NPD - NN,), _f32)])
  coefe, denom = _coef(dstp, expe, dpart, exsp)
  opart = _agg(srcp, dstp, coefe, h_groups, bn, nb)
  return opart, denom[:NN].reshape(NN, 1)


def kernel(x, edge_index, batch, W1, a_s1, a_d1, b1, W2, a_s2, a_d2, b2,
           W3, a_s3, a_d3, b3):
  del batch
  pad = jnp.zeros((EP - EE,), dtype=edge_index.dtype)
  srcp = jnp.concatenate([edge_index[0], pad])
  dstp = jnp.concatenate([edge_index[1], pad])

  h1g, asrc1, adst1, exs1 = _mm_first(x, W1, a_s1, a_d1, ng=1)
  op1, den1 = _layer_edges(srcp, dstp, h1g, asrc1, adst1, exs1, 8192, 7)

  h2g, asrc2, adst2, exs2 = _mm_next(op1, h1g, exs1, den1, b1, W2, a_s2,
                                     a_d2, ng=2)
  op2, den2 = _layer_edges(srcp, dstp, h2g, asrc2, adst2, exs2, 4096, 13)

  w3p = jnp.concatenate([W3, jnp.zeros((W3.shape[0], 14), _f32)], axis=1)
  as3p = jnp.concatenate([a_s3, jnp.zeros((14,), _f32)])
  ad3p = jnp.concatenate([a_d3, jnp.zeros((14,), _f32)])
  b3p = jnp.concatenate([b3, jnp.zeros((14,), _f32)])
  h3g, asrc3, adst3, exs3 = _mm_next(op2, h2g, exs2, den2, b2, w3p, as3p,
                                     ad3p, ng=1)
  h3 = h3g[0]
  expe3, dpart3 = _edge_ex(srcp, dstp, asrc3.reshape(-1), adst3.reshape(-1))
  exsp3 = jnp.concatenate([exs3.reshape(-1), jnp.zeros((NPD - NN,), _f32)])
  coef3, den3 = _coef(dstp, expe3, dpart3, exsp3)
  h3c0 = h3[:, 0]
  h3c1 = h3[:, 1]
  p0, p1 = _agg_pair(srcp, dstp, coef3, h3c0, h3c1)
  l0 = (p0[:NPD] + p0[NPD:])[:NN].reshape(NN, 1)
  l1 = (p1[:NPD] + p1[NPD:])[:NN].reshape(NN, 1)

  return _fin(l0, l1, h3c0.reshape(NN, 1), h3c1.reshape(NN, 1), exs3,
              den3[:NN].reshape(NN, 1), b3)
